# baseline (speedup 1.0000x reference)
import jax
import jax.numpy as jnp
from jax.experimental import pallas as pl

_W = 4
_K = 12
_TW = 253
_IMIN = -2147483648


_R = 2048


def _body(x_ref, o_ref):
    xv = x_ref[...].reshape(_R, 256)
    w = xv
    w += jnp.concatenate([xv[:, 1:], xv[:, :1]], axis=1)
    w += jnp.concatenate([xv[:, 2:], xv[:, :2]], axis=1)
    w += jnp.concatenate([xv[:, 3:], xv[:, :3]], axis=1)
    bits = jax.lax.bitcast_convert_type(w, jnp.int32)
    key = jnp.where(bits < 0, bits ^ 0x7FFFFFFF, bits)
    t = jax.lax.broadcasted_iota(jnp.int32, (_R, 256), 1)
    key = (key & -256) | t
    key = jnp.where(t < _TW, key, _IMIN)
    kt = key.T

    acc = jnp.zeros((1, _R), jnp.float32)
    for _ in range(_K):
        m = jnp.max(kt, axis=0, keepdims=True)
        kv = m & -256
        vbits = jnp.where(kv < 0, kv ^ 0x7FFFFFFF, kv)
        acc += jax.lax.bitcast_convert_type(vbits, jnp.float32)
        kt = jnp.where(kt == m, _IMIN, kt)
    zz = jnp.mean(acc.reshape(_R // 128, 128), axis=1) * (1.0 / (_K * _W))
    o_ref[0] = jnp.broadcast_to(zz[:, None], (_R // 128, 128))


@jax.jit
def kernel(x):
    B, C, F, T = x.shape
    ntile = _R // F
    xr = x.reshape(B * C // ntile, ntile * F, T)
    out = pl.pallas_call(
        _body,
        grid=(B * C // ntile,),
        in_specs=[pl.BlockSpec((1, ntile * F, T), lambda i: (i, 0, 0))],
        out_specs=pl.BlockSpec((1, ntile, 128), lambda i: (i, 0, 0)),
        out_shape=jax.ShapeDtypeStruct((B * C // ntile, ntile, 128), jnp.float32),
    )(xr)
    return out[:, :, 0].reshape(B, C)


# speedup vs baseline: 1.1599x; 1.1599x over previous
import jax
import jax.numpy as jnp
from jax.experimental import pallas as pl

_W = 4
_K = 12
_TW = 253
_IMIN = -2147483648


_R = 1024


def _body(x_ref, o_ref):
    xv = x_ref[...].reshape(_R, 256)
    w = xv
    w += jnp.concatenate([xv[:, 1:], xv[:, :1]], axis=1)
    w += jnp.concatenate([xv[:, 2:], xv[:, :2]], axis=1)
    w += jnp.concatenate([xv[:, 3:], xv[:, :3]], axis=1)
    bits = jax.lax.bitcast_convert_type(w, jnp.int32)
    key = jnp.where(bits < 0, bits ^ 0x7FFFFFFF, bits)
    t = jax.lax.broadcasted_iota(jnp.int32, (_R, 256), 1)
    key = (key & -256) | t
    key = jnp.where(t < _TW, key, _IMIN)
    kt = key.T

    v = kt.reshape(16, 2, 8, _R)
    a, b = v[:, 0], v[:, 1]
    p = jnp.maximum(a, b)
    mn = jnp.minimum(a, b)
    acc = jnp.zeros((1, 1, _R), jnp.float32)
    for _ in range(_K):
        m = jnp.max(p, axis=(0, 1), keepdims=True)
        kv = m & -256
        vbits = jnp.where(kv < 0, kv ^ 0x7FFFFFFF, kv)
        acc += jax.lax.bitcast_convert_type(vbits, jnp.float32)
        eq = p == m
        p = jnp.where(eq, mn, p)
        mn = jnp.where(eq, _IMIN, mn)
    zz = jnp.mean(acc.reshape(_R // 128, 128), axis=1) * (1.0 / (_K * _W))

    o_ref[0] = jnp.broadcast_to(zz[:, None], (_R // 128, 128))


@jax.jit
def kernel(x):
    B, C, F, T = x.shape
    ntile = _R // F
    xr = x.reshape(B * C // ntile, ntile * F, T)
    out = pl.pallas_call(
        _body,
        grid=(B * C // ntile,),
        in_specs=[pl.BlockSpec((1, ntile * F, T), lambda i: (i, 0, 0))],
        out_specs=pl.BlockSpec((1, ntile, 128), lambda i: (i, 0, 0)),
        out_shape=jax.ShapeDtypeStruct((B * C // ntile, ntile, 128), jnp.float32),
    )(xr)
    return out[:, :, 0].reshape(B, C)
